# initial kernel scaffold (unmeasured)
import jax
import jax.numpy as jnp
from jax import lax
from jax.experimental import pallas as pl
from jax.experimental.pallas import tpu as pltpu

N_Z = 4
EPS = 1e-6


def kernel(partial, resid, gamma):
    m, d = resid.shape
    mc = m // N_Z

    def body(p_ref, r_ref, g_ref, out_ref, comm_ref, send_sems, recv_sems):
        my_x = lax.axis_index("x")
        my_y = lax.axis_index("y")
        my_z = lax.axis_index("z")
        right = (my_z + 1) % N_Z
        left = (my_z + N_Z - 1) % N_Z

        def local_chunk_bf16(c):
            return p_ref[0, pl.ds(c * mc, mc), :].astype(jnp.bfloat16)

        barrier_sem = pltpu.get_barrier_semaphore()
        for nbr in [left, right]:
            pl.semaphore_signal(
                barrier_sem, inc=1,
                device_id=(my_x, my_y, nbr),
                device_id_type=pl.DeviceIdType.MESH,
            )
        pl.semaphore_wait(barrier_sem, 2)

        comm_ref[0, :, :] = local_chunk_bf16(my_z)
        for s in range(N_Z - 1):
            send_slot = s % 2
            recv_slot = (s + 1) % 2
            rdma = pltpu.make_async_remote_copy(
                src_ref=comm_ref.at[send_slot],
                dst_ref=comm_ref.at[recv_slot],
                send_sem=send_sems.at[s],
                recv_sem=recv_sems.at[s],
                device_id=(my_x, my_y, right),
                device_id_type=pl.DeviceIdType.MESH,
            )
            rdma.start()
            rdma.wait()
            c = (my_z + (N_Z - 1) - s) % N_Z
            comm_ref[recv_slot, :, :] = (
                comm_ref[recv_slot, :, :] + local_chunk_bf16(c)
            )

        o = (my_z + 1) % N_Z
        y = comm_ref[1, :, :].astype(jnp.float32) + r_ref[pl.ds(o * mc, mc), :]
        rms = jnp.sqrt(jnp.mean(y * y, axis=-1, keepdims=True) + EPS)
        outc = (y / rms) * g_ref[0, :].astype(jnp.float32)[None, :]
        out_ref[pl.ds(o * mc, mc), :] = outc
        comm_ref[0, :, :] = outc.astype(jnp.bfloat16)

        for h in range(N_Z - 1):
            s = (N_Z - 1) + h
            send_slot = h % 2
            recv_slot = (h + 1) % 2
            rdma = pltpu.make_async_remote_copy(
                src_ref=comm_ref.at[send_slot],
                dst_ref=comm_ref.at[recv_slot],
                send_sem=send_sems.at[s],
                recv_sem=recv_sems.at[s],
                device_id=(my_x, my_y, right),
                device_id_type=pl.DeviceIdType.MESH,
            )
            rdma.start()
            rdma.wait()
            c = (my_z + N_Z - h) % N_Z
            out_ref[pl.ds(c * mc, mc), :] = (
                comm_ref[recv_slot, :, :].astype(jnp.float32)
            )

    n_hops = 2 * (N_Z - 1)
    return pl.pallas_call(
        body,
        out_shape=jax.ShapeDtypeStruct((m, d), jnp.float32),
        in_specs=[
            pl.BlockSpec(memory_space=pltpu.VMEM),
            pl.BlockSpec(memory_space=pltpu.VMEM),
            pl.BlockSpec(memory_space=pltpu.VMEM),
        ],
        out_specs=pl.BlockSpec(memory_space=pltpu.VMEM),
        scratch_shapes=[
            pltpu.VMEM((2, mc, d), jnp.bfloat16),
            pltpu.SemaphoreType.DMA((n_hops,)),
            pltpu.SemaphoreType.DMA((n_hops,)),
        ],
        compiler_params=pltpu.CompilerParams(collective_id=0),
    )(partial, resid, gamma.reshape(1, d))


# baseline (device time: 165924 ns/iter reference)
import jax
import jax.numpy as jnp
from jax import lax
from jax.experimental import pallas as pl
from jax.experimental.pallas import tpu as pltpu

N_Z = 4
EPS = 1e-6


def kernel(partial, resid, gamma):
    m, d = resid.shape
    mc = m // N_Z

    def body(p_ref, r_ref, g_ref, out_ref,
             stage, comm_ref, obuf, ld_sem, st_sem, send_sems, recv_sems):
        my_x = lax.axis_index("x")
        my_y = lax.axis_index("y")
        my_z = lax.axis_index("z")
        right = (my_z + 1) % N_Z
        left = (my_z + N_Z - 1) % N_Z

        def load_partial_chunk(c):
            cp = pltpu.make_async_copy(
                p_ref.at[0, pl.ds(c * mc, mc), :], stage, ld_sem)
            cp.start()
            cp.wait()

        barrier_sem = pltpu.get_barrier_semaphore()
        for nbr in [left, right]:
            pl.semaphore_signal(
                barrier_sem, inc=1,
                device_id=(my_x, my_y, nbr),
                device_id_type=pl.DeviceIdType.MESH,
            )
        pl.semaphore_wait(barrier_sem, 2)

        load_partial_chunk(my_z)
        comm_ref[0, :, :] = stage[:, :].astype(jnp.bfloat16)
        for s in range(N_Z - 1):
            send_slot = s % 2
            recv_slot = (s + 1) % 2
            rdma = pltpu.make_async_remote_copy(
                src_ref=comm_ref.at[send_slot],
                dst_ref=comm_ref.at[recv_slot],
                send_sem=send_sems.at[s],
                recv_sem=recv_sems.at[s],
                device_id=(my_x, my_y, right),
                device_id_type=pl.DeviceIdType.MESH,
            )
            rdma.start()
            c = (my_z + (N_Z - 1) - s) % N_Z
            load_partial_chunk(c)
            rdma.wait()
            comm_ref[recv_slot, :, :] = (
                comm_ref[recv_slot, :, :] + stage[:, :].astype(jnp.bfloat16)
            )

        o = (my_z + 1) % N_Z
        cp = pltpu.make_async_copy(r_ref.at[pl.ds(o * mc, mc), :], stage, ld_sem)
        cp.start()
        cp.wait()
        y = comm_ref[1, :, :].astype(jnp.float32) + stage[:, :]
        rms = jnp.sqrt(jnp.mean(y * y, axis=-1, keepdims=True) + EPS)
        outc = (y / rms) * g_ref[0, :][None, :]
        obuf[:, :] = outc
        st = pltpu.make_async_copy(obuf, out_ref.at[pl.ds(o * mc, mc), :], st_sem)
        st.start()
        comm_ref[0, :, :] = outc.astype(jnp.bfloat16)

        for h in range(N_Z - 1):
            s = (N_Z - 1) + h
            send_slot = h % 2
            recv_slot = (h + 1) % 2
            rdma = pltpu.make_async_remote_copy(
                src_ref=comm_ref.at[send_slot],
                dst_ref=comm_ref.at[recv_slot],
                send_sem=send_sems.at[s],
                recv_sem=recv_sems.at[s],
                device_id=(my_x, my_y, right),
                device_id_type=pl.DeviceIdType.MESH,
            )
            rdma.start()
            rdma.wait()
            st.wait()
            c = (my_z + N_Z - h) % N_Z
            obuf[:, :] = comm_ref[recv_slot, :, :].astype(jnp.float32)
            st = pltpu.make_async_copy(
                obuf, out_ref.at[pl.ds(c * mc, mc), :], st_sem)
            st.start()
        st.wait()

    n_hops = 2 * (N_Z - 1)
    return pl.pallas_call(
        body,
        out_shape=jax.ShapeDtypeStruct((m, d), jnp.float32),
        in_specs=[
            pl.BlockSpec(memory_space=pl.ANY),
            pl.BlockSpec(memory_space=pl.ANY),
            pl.BlockSpec(memory_space=pltpu.VMEM),
        ],
        out_specs=pl.BlockSpec(memory_space=pl.ANY),
        scratch_shapes=[
            pltpu.VMEM((mc, d), jnp.float32),
            pltpu.VMEM((2, mc, d), jnp.bfloat16),
            pltpu.VMEM((mc, d), jnp.float32),
            pltpu.SemaphoreType.DMA,
            pltpu.SemaphoreType.DMA,
            pltpu.SemaphoreType.DMA((n_hops,)),
            pltpu.SemaphoreType.DMA((n_hops,)),
        ],
        compiler_params=pltpu.CompilerParams(collective_id=0),
    )(partial, resid, gamma.reshape(1, d))


# device time: 99509 ns/iter; 1.6674x vs baseline; 1.6674x over previous
import os

import jax
import jax.numpy as jnp
from jax import lax
from jax.experimental import pallas as pl
from jax.experimental.pallas import tpu as pltpu

N_Z = 4
EPS = 1e-6
DEBUG_STAGE = int(os.environ.get("DEBUG_STAGE", "4"))


def kernel(partial, resid, gamma):
    m, d = resid.shape
    mq = m // 4
    mp = mq // N_Z

    def body(p_ref, r_ref, g_ref, out_ref,
             stage, comm_ref, allq, opiece, ostrip, ysend, diagbuf,
             ld_sem, stp_sems, sts_sems,
             rs_ssem, rs_rsem, zag_ssem, zag_rsem,
             x_ssem, x_rsem, y_ssem, y_rsem):
        my_x = lax.axis_index("x")
        my_y = lax.axis_index("y")
        my_z = lax.axis_index("z")
        right = (my_z + 1) % N_Z
        left = (my_z + N_Z - 1) % N_Z
        q = 2 * my_x + my_y
        qx = 2 * (1 - my_x) + my_y
        qy = 2 * my_x + (1 - my_y)
        qd = 2 * (1 - my_x) + (1 - my_y)

        def piece_rows(quarter, j):
            return pl.ds(quarter * mq + j * mp, mp)

        def load_partial_piece(j):
            cp = pltpu.make_async_copy(
                p_ref.at[0, piece_rows(q, j), :], stage, ld_sem)
            cp.start()
            cp.wait()

        barrier_sem = pltpu.get_barrier_semaphore()
        for dev in [(my_x, my_y, left), (my_x, my_y, right),
                    (1 - my_x, my_y, my_z), (my_x, 1 - my_y, my_z)]:
            pl.semaphore_signal(
                barrier_sem, inc=1,
                device_id=dev, device_id_type=pl.DeviceIdType.MESH,
            )
        pl.semaphore_wait(barrier_sem, 4)

        load_partial_piece(my_z)
        comm_ref[0, :, :] = stage[:, :].astype(jnp.bfloat16)
        for s in range(N_Z - 1):
            rdma = pltpu.make_async_remote_copy(
                src_ref=comm_ref.at[s % 2],
                dst_ref=comm_ref.at[(s + 1) % 2],
                send_sem=rs_ssem.at[s],
                recv_sem=rs_rsem.at[s],
                device_id=(my_x, my_y, right),
                device_id_type=pl.DeviceIdType.MESH,
            )
            rdma.start()
            load_partial_piece((my_z + (N_Z - 1) - s) % N_Z)
            rdma.wait()
            comm_ref[(s + 1) % 2, :, :] = (
                comm_ref[(s + 1) % 2, :, :] + stage[:, :].astype(jnp.bfloat16)
            )

        j_own = (my_z + 1) % N_Z
        cp = pltpu.make_async_copy(
            r_ref.at[piece_rows(q, j_own), :], stage, ld_sem)
        cp.start()
        cp.wait()
        y = comm_ref[1, :, :].astype(jnp.float32) + stage[:, :]
        rms = jnp.sqrt(jnp.mean(y * y, axis=-1, keepdims=True) + EPS)
        outc = (y / rms) * g_ref[0, :][None, :]
        allq[piece_rows(q, j_own), :] = outc.astype(jnp.bfloat16)
        opiece[0, :, :] = outc
        stp = pltpu.make_async_copy(
            opiece.at[0], out_ref.at[piece_rows(q, j_own), :], stp_sems.at[0])
        stp.start()
        stp_pending = [stp, None]

        if DEBUG_STAGE < 2:
            for stp in stp_pending:
                if stp is not None:
                    stp.wait()
            return

        for h in range(N_Z - 1):
            js = (my_z + 1 - h + N_Z) % N_Z
            jr = (my_z - h + N_Z) % N_Z
            rdma = pltpu.make_async_remote_copy(
                src_ref=allq.at[piece_rows(q, js), :],
                dst_ref=allq.at[piece_rows(q, js), :],
                send_sem=zag_ssem.at[h],
                recv_sem=zag_rsem.at[h],
                device_id=(my_x, my_y, right),
                device_id_type=pl.DeviceIdType.MESH,
            )
            rdma.start()
            rdma.wait()
            slot = (h + 1) % 2
            if stp_pending[slot] is not None:
                stp_pending[slot].wait()
            opiece[slot, :, :] = allq[piece_rows(q, jr), :].astype(jnp.float32)
            stp = pltpu.make_async_copy(
                opiece.at[slot], out_ref.at[piece_rows(q, jr), :],
                stp_sems.at[slot])
            stp.start()
            stp_pending[slot] = stp

        if DEBUG_STAGE < 3:
            for stp in stp_pending:
                if stp is not None:
                    stp.wait()
            return

        x_swap = pltpu.make_async_remote_copy(
            src_ref=allq.at[pl.ds(q * mq, mq), :],
            dst_ref=allq.at[pl.ds(q * mq, mq), :],
            send_sem=x_ssem.at[0], recv_sem=x_rsem.at[0],
            device_id=(1 - my_x, my_y, my_z),
            device_id_type=pl.DeviceIdType.MESH,
        )
        y_swap = pltpu.make_async_remote_copy(
            src_ref=allq.at[pl.ds(q * mq, mq), :],
            dst_ref=allq.at[pl.ds(q * mq, mq), :],
            send_sem=y_ssem.at[0], recv_sem=y_rsem.at[0],
            device_id=(my_x, 1 - my_y, my_z),
            device_id_type=pl.DeviceIdType.MESH,
        )
        x_swap.start()
        y_swap.start()
        x_swap.wait()
        y_swap.wait()

        sts_pending = [None, None]
        for slot, quarter in [(0, qx), (1, qy)]:
            ostrip[slot, :, :] = allq[pl.ds(quarter * mq, mq), :].astype(
                jnp.float32)
            sts = pltpu.make_async_copy(
                ostrip.at[slot], out_ref.at[pl.ds(quarter * mq, mq), :],
                sts_sems.at[slot])
            sts.start()
            sts_pending[slot] = sts

        if DEBUG_STAGE < 4:
            for pend in sts_pending + stp_pending:
                if pend is not None:
                    pend.wait()
            return

        x_diag = pltpu.make_async_remote_copy(
            src_ref=allq.at[pl.ds(qy * mq, 2 * mp), :],
            dst_ref=allq.at[pl.ds(qy * mq, 2 * mp), :],
            send_sem=x_ssem.at[1], recv_sem=x_rsem.at[1],
            device_id=(1 - my_x, my_y, my_z),
            device_id_type=pl.DeviceIdType.MESH,
        )
        ysend[:, :] = allq[pl.ds(qx * mq + 2 * mp, 2 * mp), :]
        y_diag = pltpu.make_async_remote_copy(
            src_ref=ysend,
            dst_ref=diagbuf,
            send_sem=y_ssem.at[1], recv_sem=y_rsem.at[1],
            device_id=(my_x, 1 - my_y, my_z),
            device_id_type=pl.DeviceIdType.MESH,
        )
        x_diag.start()
        y_diag.start()
        x_diag.wait()
        y_diag.wait()
        allq[pl.ds(qd * mq + 2 * mp, 2 * mp), :] = diagbuf[:, :]

        sts_pending[0].wait()
        ostrip[0, :, :] = allq[pl.ds(qd * mq, mq), :].astype(jnp.float32)
        sts = pltpu.make_async_copy(
            ostrip.at[0], out_ref.at[pl.ds(qd * mq, mq), :], sts_sems.at[0])
        sts.start()
        sts.wait()
        sts_pending[1].wait()
        for stp in stp_pending:
            if stp is not None:
                stp.wait()

    return pl.pallas_call(
        body,
        out_shape=jax.ShapeDtypeStruct((m, d), jnp.float32),
        in_specs=[
            pl.BlockSpec(memory_space=pl.ANY),
            pl.BlockSpec(memory_space=pl.ANY),
            pl.BlockSpec(memory_space=pltpu.VMEM),
        ],
        out_specs=pl.BlockSpec(memory_space=pl.ANY),
        scratch_shapes=[
            pltpu.VMEM((mp, d), jnp.float32),
            pltpu.VMEM((2, mp, d), jnp.bfloat16),
            pltpu.VMEM((m, d), jnp.bfloat16),
            pltpu.VMEM((2, mp, d), jnp.float32),
            pltpu.VMEM((2, mq, d), jnp.float32),
            pltpu.VMEM((2 * mp, d), jnp.bfloat16),
            pltpu.VMEM((2 * mp, d), jnp.bfloat16),
            pltpu.SemaphoreType.DMA,
            pltpu.SemaphoreType.DMA((2,)),
            pltpu.SemaphoreType.DMA((2,)),
            pltpu.SemaphoreType.DMA((N_Z - 1,)),
            pltpu.SemaphoreType.DMA((N_Z - 1,)),
            pltpu.SemaphoreType.DMA((N_Z - 1,)),
            pltpu.SemaphoreType.DMA((N_Z - 1,)),
            pltpu.SemaphoreType.DMA((2,)),
            pltpu.SemaphoreType.DMA((2,)),
            pltpu.SemaphoreType.DMA((2,)),
            pltpu.SemaphoreType.DMA((2,)),
        ],
        compiler_params=pltpu.CompilerParams(collective_id=0),
    )(partial, resid, gamma.reshape(1, d))


# device time: 81396 ns/iter; 2.0385x vs baseline; 1.2225x over previous
import jax
import jax.numpy as jnp
from jax import lax
from jax.experimental import pallas as pl
from jax.experimental.pallas import tpu as pltpu

N_Z = 4
EPS = 1e-6


def kernel(partial, resid, gamma):
    m, d = resid.shape
    mq = m // 4
    mp = mq // N_Z

    def body(p_ref, r_ref, g_ref, out_ref,
             stage, comm_ref, allq, opiece, ostrip, ysend, diagbuf,
             ld_sem, stp_sems, sts_sems,
             rs_ssem, rs_rsem, zag_ssem, zag_rsem,
             xp_ssem, xp_rsem, yp_ssem, yp_rsem,
             xd_ssem, xd_rsem, yd_ssem, yd_rsem):
        my_x = lax.axis_index("x")
        my_y = lax.axis_index("y")
        my_z = lax.axis_index("z")
        right = (my_z + 1) % N_Z
        left = (my_z + N_Z - 1) % N_Z
        q = 2 * my_x + my_y
        qx = 2 * (1 - my_x) + my_y
        qy = 2 * my_x + (1 - my_y)
        qd = 2 * (1 - my_x) + (1 - my_y)

        def piece_rows(quarter, j):
            return pl.ds(quarter * mq + j * mp, mp)

        def load_partial_piece(j):
            cp = pltpu.make_async_copy(
                p_ref.at[0, piece_rows(q, j), :], stage, ld_sem)
            cp.start()
            cp.wait()

        barrier_sem = pltpu.get_barrier_semaphore()
        for dev in [(my_x, my_y, left), (my_x, my_y, right),
                    (1 - my_x, my_y, my_z), (my_x, 1 - my_y, my_z)]:
            pl.semaphore_signal(
                barrier_sem, inc=1,
                device_id=dev, device_id_type=pl.DeviceIdType.MESH,
            )
        pl.semaphore_wait(barrier_sem, 4)

        load_partial_piece(my_z)
        comm_ref[0, :, :] = stage[:, :].astype(jnp.bfloat16)
        for s in range(N_Z - 1):
            rdma = pltpu.make_async_remote_copy(
                src_ref=comm_ref.at[s % 2],
                dst_ref=comm_ref.at[(s + 1) % 2],
                send_sem=rs_ssem.at[s],
                recv_sem=rs_rsem.at[s],
                device_id=(my_x, my_y, right),
                device_id_type=pl.DeviceIdType.MESH,
            )
            rdma.start()
            load_partial_piece((my_z + (N_Z - 1) - s) % N_Z)
            rdma.wait()
            comm_ref[(s + 1) % 2, :, :] = (
                comm_ref[(s + 1) % 2, :, :] + stage[:, :].astype(jnp.bfloat16)
            )

        j_own = (my_z + 1) % N_Z
        cp = pltpu.make_async_copy(
            r_ref.at[piece_rows(q, j_own), :], stage, ld_sem)
        cp.start()
        cp.wait()
        y = comm_ref[1, :, :].astype(jnp.float32) + stage[:, :]
        rms = jnp.sqrt(jnp.mean(y * y, axis=-1, keepdims=True) + EPS)
        outc = (y / rms) * g_ref[0, :][None, :]
        allq[piece_rows(q, j_own), :] = outc.astype(jnp.bfloat16)

        def plane_send(j, i):
            sends = []
            for ssem, rsem, dev in [
                (xp_ssem, xp_rsem, (1 - my_x, my_y, my_z)),
                (yp_ssem, yp_rsem, (my_x, 1 - my_y, my_z)),
            ]:
                rdma = pltpu.make_async_remote_copy(
                    src_ref=allq.at[piece_rows(q, j), :],
                    dst_ref=allq.at[piece_rows(q, j), :],
                    send_sem=ssem.at[i],
                    recv_sem=rsem.at[i],
                    device_id=dev,
                    device_id_type=pl.DeviceIdType.MESH,
                )
                rdma.start()
                sends.append(rdma)
            return sends

        plane_sends = plane_send(j_own, 0)

        opiece[0, :, :] = outc
        stp = pltpu.make_async_copy(
            opiece.at[0], out_ref.at[piece_rows(q, j_own), :], stp_sems.at[0])
        stp.start()
        stp_pending = [stp, None]

        for h in range(N_Z - 1):
            js = (my_z + 1 - h + N_Z) % N_Z
            jr = (my_z - h + N_Z) % N_Z
            rdma = pltpu.make_async_remote_copy(
                src_ref=allq.at[piece_rows(q, js), :],
                dst_ref=allq.at[piece_rows(q, js), :],
                send_sem=zag_ssem.at[h],
                recv_sem=zag_rsem.at[h],
                device_id=(my_x, my_y, right),
                device_id_type=pl.DeviceIdType.MESH,
            )
            rdma.start()
            rdma.wait()
            plane_sends += plane_send(jr, h + 1)
            slot = (h + 1) % 2
            if stp_pending[slot] is not None:
                stp_pending[slot].wait()
            opiece[slot, :, :] = allq[piece_rows(q, jr), :].astype(jnp.float32)
            stp = pltpu.make_async_copy(
                opiece.at[slot], out_ref.at[piece_rows(q, jr), :],
                stp_sems.at[slot])
            stp.start()
            stp_pending[slot] = stp

        def wait_plane_recvs(rsem, quarter):
            for i in range(N_Z):
                recv = pltpu.make_async_remote_copy(
                    src_ref=allq.at[piece_rows(quarter, i), :],
                    dst_ref=allq.at[piece_rows(quarter, i), :],
                    send_sem=rsem.at[i],
                    recv_sem=rsem.at[i],
                    device_id=(my_x, my_y, my_z),
                    device_id_type=pl.DeviceIdType.MESH,
                )
                recv.wait_recv()

        sts_pending = [None, None]

        wait_plane_recvs(yp_rsem, qy)
        x_diag = pltpu.make_async_remote_copy(
            src_ref=allq.at[pl.ds(qy * mq, 2 * mp), :],
            dst_ref=allq.at[pl.ds(qy * mq, 2 * mp), :],
            send_sem=xd_ssem, recv_sem=xd_rsem,
            device_id=(1 - my_x, my_y, my_z),
            device_id_type=pl.DeviceIdType.MESH,
        )
        x_diag.start()
        ostrip[1, :, :] = allq[pl.ds(qy * mq, mq), :].astype(jnp.float32)
        sts = pltpu.make_async_copy(
            ostrip.at[1], out_ref.at[pl.ds(qy * mq, mq), :], sts_sems.at[1])
        sts.start()
        sts_pending[1] = sts

        wait_plane_recvs(xp_rsem, qx)
        ysend[:, :] = allq[pl.ds(qx * mq + 2 * mp, 2 * mp), :]
        y_diag = pltpu.make_async_remote_copy(
            src_ref=ysend,
            dst_ref=diagbuf,
            send_sem=yd_ssem, recv_sem=yd_rsem,
            device_id=(my_x, 1 - my_y, my_z),
            device_id_type=pl.DeviceIdType.MESH,
        )
        y_diag.start()
        ostrip[0, :, :] = allq[pl.ds(qx * mq, mq), :].astype(jnp.float32)
        sts = pltpu.make_async_copy(
            ostrip.at[0], out_ref.at[pl.ds(qx * mq, mq), :], sts_sems.at[0])
        sts.start()
        sts_pending[0] = sts

        x_diag.wait()
        y_diag.wait()
        allq[pl.ds(qd * mq + 2 * mp, 2 * mp), :] = diagbuf[:, :]
        sts_pending[0].wait()
        ostrip[0, :, :] = allq[pl.ds(qd * mq, mq), :].astype(jnp.float32)
        sts = pltpu.make_async_copy(
            ostrip.at[0], out_ref.at[pl.ds(qd * mq, mq), :], sts_sems.at[0])
        sts.start()
        sts.wait()
        sts_pending[1].wait()
        for pend in stp_pending:
            if pend is not None:
                pend.wait()
        for snd in plane_sends:
            snd.wait_send()

    return pl.pallas_call(
        body,
        out_shape=jax.ShapeDtypeStruct((m, d), jnp.float32),
        in_specs=[
            pl.BlockSpec(memory_space=pl.ANY),
            pl.BlockSpec(memory_space=pl.ANY),
            pl.BlockSpec(memory_space=pltpu.VMEM),
        ],
        out_specs=pl.BlockSpec(memory_space=pl.ANY),
        scratch_shapes=[
            pltpu.VMEM((mp, d), jnp.float32),
            pltpu.VMEM((2, mp, d), jnp.bfloat16),
            pltpu.VMEM((m, d), jnp.bfloat16),
            pltpu.VMEM((2, mp, d), jnp.float32),
            pltpu.VMEM((2, mq, d), jnp.float32),
            pltpu.VMEM((2 * mp, d), jnp.bfloat16),
            pltpu.VMEM((2 * mp, d), jnp.bfloat16),
            pltpu.SemaphoreType.DMA,
            pltpu.SemaphoreType.DMA((2,)),
            pltpu.SemaphoreType.DMA((2,)),
            pltpu.SemaphoreType.DMA((N_Z - 1,)),
            pltpu.SemaphoreType.DMA((N_Z - 1,)),
            pltpu.SemaphoreType.DMA((N_Z - 1,)),
            pltpu.SemaphoreType.DMA((N_Z - 1,)),
            pltpu.SemaphoreType.DMA((N_Z,)),
            pltpu.SemaphoreType.DMA((N_Z,)),
            pltpu.SemaphoreType.DMA((N_Z,)),
            pltpu.SemaphoreType.DMA((N_Z,)),
            pltpu.SemaphoreType.DMA,
            pltpu.SemaphoreType.DMA,
            pltpu.SemaphoreType.DMA,
            pltpu.SemaphoreType.DMA,
        ],
        compiler_params=pltpu.CompilerParams(collective_id=0),
    )(partial, resid, gamma.reshape(1, d))


# device time: 78735 ns/iter; 2.1074x vs baseline; 1.0338x over previous
import jax
import jax.numpy as jnp
from jax import lax
from jax.experimental import pallas as pl
from jax.experimental.pallas import tpu as pltpu

N_Z = 4
EPS = 1e-6


def kernel(partial, resid, gamma):
    m, d = resid.shape
    mq = m // 4
    mp = mq // N_Z

    def body(p_ref, r_ref, g_ref, out_ref,
             stage, comm_ref, allq, ostrip,
             ld_sem, sts_sems,
             rs_ssem, rs_rsem, zag_ssem, zag_rsem,
             xp_ssem, xp_rsem, yp_ssem, yp_rsem,
             xd_ssem, xd_rsem, yd_ssem, yd_rsem):
        my_x = lax.axis_index("x")
        my_y = lax.axis_index("y")
        my_z = lax.axis_index("z")
        right = (my_z + 1) % N_Z
        left = (my_z + N_Z - 1) % N_Z
        q = 2 * my_x + my_y
        qx = 2 * (1 - my_x) + my_y
        qy = 2 * my_x + (1 - my_y)
        qd = 2 * (1 - my_x) + (1 - my_y)
        x_nbr = (1 - my_x, my_y, my_z)
        y_nbr = (my_x, 1 - my_y, my_z)

        def piece_rows(quarter, j):
            return pl.ds(quarter * mq + j * mp, mp)

        def load_partial_piece(j):
            cp = pltpu.make_async_copy(
                p_ref.at[0, piece_rows(q, j), :], stage, ld_sem)
            cp.start()
            cp.wait()

        def piece_rdma(quarter, j, ssem, rsem, dev):
            rdma = pltpu.make_async_remote_copy(
                src_ref=allq.at[piece_rows(quarter, j), :],
                dst_ref=allq.at[piece_rows(quarter, j), :],
                send_sem=ssem, recv_sem=rsem,
                device_id=dev, device_id_type=pl.DeviceIdType.MESH,
            )
            rdma.start()
            return rdma

        load_partial_piece(my_z)
        comm_ref[0, :, :] = stage[:, :].astype(jnp.bfloat16)

        barrier_sem = pltpu.get_barrier_semaphore()
        for dev in [(my_x, my_y, left), (my_x, my_y, right), x_nbr, y_nbr]:
            pl.semaphore_signal(
                barrier_sem, inc=1,
                device_id=dev, device_id_type=pl.DeviceIdType.MESH,
            )
        pl.semaphore_wait(barrier_sem, 4)

        for s in range(N_Z - 1):
            rdma = pltpu.make_async_remote_copy(
                src_ref=comm_ref.at[s % 2],
                dst_ref=comm_ref.at[(s + 1) % 2],
                send_sem=rs_ssem.at[s],
                recv_sem=rs_rsem.at[s],
                device_id=(my_x, my_y, right),
                device_id_type=pl.DeviceIdType.MESH,
            )
            rdma.start()
            load_partial_piece((my_z + (N_Z - 1) - s) % N_Z)
            rdma.wait()
            comm_ref[(s + 1) % 2, :, :] = (
                comm_ref[(s + 1) % 2, :, :] + stage[:, :].astype(jnp.bfloat16)
            )

        j_own = (my_z + 1) % N_Z
        cp = pltpu.make_async_copy(
            r_ref.at[piece_rows(q, j_own), :], stage, ld_sem)
        cp.start()
        cp.wait()
        y = comm_ref[1, :, :].astype(jnp.float32) + stage[:, :]
        rms = jnp.sqrt(jnp.mean(y * y, axis=-1, keepdims=True) + EPS)
        outc = (y / rms) * g_ref[0, :][None, :]
        allq[piece_rows(q, j_own), :] = outc.astype(jnp.bfloat16)

        sends = [
            piece_rdma(q, j_own, xp_ssem.at[0], xp_rsem.at[0], x_nbr),
            piece_rdma(q, j_own, yp_ssem.at[0], yp_rsem.at[0], y_nbr),
        ]

        for h in range(N_Z - 1):
            js = (my_z + 1 - h + N_Z) % N_Z
            jr = (my_z - h + N_Z) % N_Z
            rdma = pltpu.make_async_remote_copy(
                src_ref=allq.at[piece_rows(q, js), :],
                dst_ref=allq.at[piece_rows(q, js), :],
                send_sem=zag_ssem.at[h],
                recv_sem=zag_rsem.at[h],
                device_id=(my_x, my_y, right),
                device_id_type=pl.DeviceIdType.MESH,
            )
            rdma.start()
            rdma.wait()
            i = h + 1
            sends.append(piece_rdma(q, jr, xp_ssem.at[i], xp_rsem.at[i], x_nbr))
            sends.append(piece_rdma(q, jr, yp_ssem.at[i], yp_rsem.at[i], y_nbr))

        def strip_store(quarter, slot):
            ostrip[slot, :, :] = allq[pl.ds(quarter * mq, mq), :].astype(
                jnp.float32)
            sts = pltpu.make_async_copy(
                ostrip.at[slot], out_ref.at[pl.ds(quarter * mq, mq), :],
                sts_sems.at[slot])
            sts.start()
            return sts

        sts_pending = [strip_store(q, 0), None]

        def wait_recv(quarter, i, rsem):
            recv = pltpu.make_async_remote_copy(
                src_ref=allq.at[piece_rows(quarter, 0), :],
                dst_ref=allq.at[piece_rows(quarter, 0), :],
                send_sem=rsem, recv_sem=rsem,
                device_id=(my_x, my_y, my_z),
                device_id_type=pl.DeviceIdType.MESH,
            )
            recv.wait_recv()
            return (quarter * 0 + my_z + 1 - i + N_Z) % N_Z

        for i in range(2):
            j = wait_recv(qy, i, yp_rsem.at[i])
            sends.append(piece_rdma(qy, j, xd_ssem.at[i], xd_rsem.at[i], x_nbr))
        for i in range(3):
            j = wait_recv(qx, i, xp_rsem.at[i])
        sends.append(piece_rdma(qx, j, yd_ssem.at[0], yd_rsem.at[0], y_nbr))
        j = wait_recv(qx, 3, xp_rsem.at[3])
        sends.append(piece_rdma(qx, j, yd_ssem.at[1], yd_rsem.at[1], y_nbr))
        sts_pending[1] = strip_store(qx, 1)
        for i in range(2, N_Z):
            wait_recv(qy, i, yp_rsem.at[i])
        sts_pending[0].wait()
        sts_pending[0] = strip_store(qy, 0)

        for i in range(2):
            wait_recv(qd, i, xd_rsem.at[i])
            wait_recv(qd, i, yd_rsem.at[i])
        sts_pending[1].wait()
        sts = strip_store(qd, 1)
        sts.wait()
        sts_pending[0].wait()
        for snd in sends:
            snd.wait_send()

    return pl.pallas_call(
        body,
        out_shape=jax.ShapeDtypeStruct((m, d), jnp.float32),
        in_specs=[
            pl.BlockSpec(memory_space=pl.ANY),
            pl.BlockSpec(memory_space=pl.ANY),
            pl.BlockSpec(memory_space=pltpu.VMEM),
        ],
        out_specs=pl.BlockSpec(memory_space=pl.ANY),
        scratch_shapes=[
            pltpu.VMEM((mp, d), jnp.float32),
            pltpu.VMEM((2, mp, d), jnp.bfloat16),
            pltpu.VMEM((m, d), jnp.bfloat16),
            pltpu.VMEM((2, mq, d), jnp.float32),
            pltpu.SemaphoreType.DMA,
            pltpu.SemaphoreType.DMA((2,)),
            pltpu.SemaphoreType.DMA((N_Z - 1,)),
            pltpu.SemaphoreType.DMA((N_Z - 1,)),
            pltpu.SemaphoreType.DMA((N_Z - 1,)),
            pltpu.SemaphoreType.DMA((N_Z - 1,)),
            pltpu.SemaphoreType.DMA((N_Z,)),
            pltpu.SemaphoreType.DMA((N_Z,)),
            pltpu.SemaphoreType.DMA((N_Z,)),
            pltpu.SemaphoreType.DMA((N_Z,)),
            pltpu.SemaphoreType.DMA((2,)),
            pltpu.SemaphoreType.DMA((2,)),
            pltpu.SemaphoreType.DMA((2,)),
            pltpu.SemaphoreType.DMA((2,)),
        ],
        compiler_params=pltpu.CompilerParams(collective_id=0),
    )(partial, resid, gamma.reshape(1, d))
